# baseline (speedup 1.0000x reference)
import functools

import jax
import jax.numpy as jnp
from jax import lax
from jax.experimental import pallas as pl
from jax.experimental.pallas import tpu as pltpu
from jax.experimental.pallas import tpu_sc as plsc

_T = 32768
_D = 64
_DP = 128
_B = 16384
_NC = 2
_NS = 16
_NW = _NC * _NS
_BPW = _B // _NW
_CHUNK = 128
_NCH = _BPW // _CHUNK
_L = 16

_mesh = plsc.VectorSubcoreMesh(core_axis_name="c", subcore_axis_name="s")


@functools.partial(
    pl.kernel,
    mesh=_mesh,
    out_type=jax.ShapeDtypeStruct((_B, _DP), jnp.float32),
    scratch_types=[
        pltpu.VMEM((_BPW,), jnp.int32),
        pltpu.VMEM((_BPW,), jnp.int32),
        pltpu.VMEM((_BPW, _DP), jnp.float32),
        pltpu.VMEM((_CHUNK, _DP), jnp.float32),
        [pltpu.SemaphoreType.DMA] * _NCH,
    ],
    compiler_params=pltpu.CompilerParams(needs_layout_passes=False),
)
def _pe_gather(x_hbm, pe_hbm, out_hbm, idx_v, lidx_v, lines_v, outv, sems):
    wid = lax.axis_index("s") * _NC + lax.axis_index("c")
    base = wid * _BPW
    pltpu.sync_copy(x_hbm.at[pl.ds(base, _BPW)], idx_v)
    for k in range(_BPW // _L):
        lidx_v[pl.ds(_L * k, _L)] = idx_v[pl.ds(_L * k, _L)] >> 1
    copies = [
        pltpu.async_copy(
            pe_hbm.at[lidx_v.at[pl.ds(j * _CHUNK, _CHUNK)]],
            lines_v.at[pl.ds(j * _CHUNK, _CHUNK)],
            sems[j],
        )
        for j in range(_NCH)
    ]
    for j, c in enumerate(copies):
        c.wait()

        def select_group(g, carry, j=j):
            b0 = j * _CHUNK + g * _L
            par = idx_v[pl.ds(b0, _L)] & 1
            for l in range(_L):
                off = par[l] * _D
                for q in range(_D // _L):
                    outv[g * _L + l, pl.ds(_L * q, _L)] = (
                        lines_v[b0 + l, pl.ds(off + _L * q, _L)])
            return carry

        lax.fori_loop(0, _CHUNK // _L, select_group, 0)
        pltpu.sync_copy(outv,
                        out_hbm.at[pl.ds(base + j * _CHUNK, _CHUNK)])


def kernel(x, pe):
    pe2 = pe.reshape(_T // 2, _DP)
    return _pe_gather(x.astype(jnp.int32), pe2)[:, :_D]


# speedup vs baseline: 1.2059x; 1.2059x over previous
import functools

import jax
import jax.numpy as jnp
from jax import lax
from jax.experimental import pallas as pl
from jax.experimental.pallas import tpu as pltpu
from jax.experimental.pallas import tpu_sc as plsc

_T = 32768
_D = 64
_DP = 128
_B = 16384
_NC = 2
_NS = 16
_NW = _NC * _NS
_BPW = _B // _NW
_CHUNK = 128
_NCH = _BPW // _CHUNK

_mesh = plsc.VectorSubcoreMesh(core_axis_name="c", subcore_axis_name="s")


@functools.partial(
    pl.kernel,
    mesh=_mesh,
    out_type=jax.ShapeDtypeStruct((_B, _DP), jnp.float32),
    scratch_types=[
        pltpu.VMEM((_BPW,), jnp.int32),
        pltpu.VMEM((_BPW, _D), jnp.float32),
        [pltpu.SemaphoreType.DMA] * _NCH,
    ],
    compiler_params=pltpu.CompilerParams(use_tc_tiling_on_sc=False),
)
def _pe_gather(x_hbm, pe_hbm, out_hbm, idx_v, rows_v, sems):
    wid = lax.axis_index("s") * _NC + lax.axis_index("c")
    base = wid * _BPW
    pltpu.sync_copy(x_hbm.at[pl.ds(base, _BPW)], idx_v)
    copies = [
        pltpu.async_copy(
            pe_hbm.at[idx_v.at[pl.ds(j * _CHUNK, _CHUNK)]],
            rows_v.at[pl.ds(j * _CHUNK, _CHUNK)],
            sems[j],
        )
        for j in range(_NCH)
    ]
    for j, c in enumerate(copies):
        c.wait()
        pltpu.sync_copy(
            rows_v.at[pl.ds(j * _CHUNK, _CHUNK)],
            out_hbm.at[pl.ds(base + j * _CHUNK, _CHUNK), pl.ds(0, _D)],
        )


def kernel(x, pe):
    return _pe_gather(x.astype(jnp.int32), pe)[:, :_D]
